# Initial kernel scaffold; baseline (speedup 1.0000x reference)
import jax
import jax.numpy as jnp
from jax.experimental import pallas as pl


def kernel(score):
    raise NotImplementedError("write your pallas kernel here")


# speedup vs baseline: 3.0465x; 3.0465x over previous
import functools

import jax
import jax.numpy as jnp
from jax import lax
from jax.experimental import pallas as pl
from jax.experimental.pallas import tpu as pltpu
from jax.experimental.pallas import tpu_sc as plsc

PATH_NUM = 64
N_TOKENS = 32768
NUM_CORES = 2
NUM_SUBCORES = 16
NUM_WORKERS = NUM_CORES * NUM_SUBCORES
ROWS_PER_WORKER = N_TOKENS // NUM_WORKERS
CHUNK_ROWS = 256
NUM_CHUNKS = ROWS_PER_WORKER // CHUNK_ROWS
LANES = 16
VPR = PATH_NUM // LANES

_NEG_INF = float("-inf")
_BIG_IDX = PATH_NUM


def _min_tree(xs):
    return jnp.minimum(jnp.minimum(xs[0], xs[1]), jnp.minimum(xs[2], xs[3]))


def _max_tree(xs):
    return jnp.maximum(jnp.maximum(xs[0], xs[1]), jnp.maximum(xs[2], xs[3]))


def _topk_body(score_hbm, out_hbm, vin, vout):
    wid = lax.axis_index("s") * NUM_CORES + lax.axis_index("c")
    iotas = [lax.iota(jnp.int32, LANES) + LANES * j for j in range(VPR)]
    one = jnp.full((LANES,), 1, jnp.int32)
    zero = jnp.full((LANES,), 0, jnp.int32)

    def row_body(r, carry):
        off = r * PATH_NUM
        vs = [vin[pl.ds(off + LANES * j, LANES)] for j in range(VPR)]
        s1 = jnp.max(_max_tree(vs))
        i1 = jnp.min(_min_tree(
            [jnp.where(vs[j] == s1, iotas[j], _BIG_IDX) for j in range(VPR)]))
        eq1 = [iotas[j] == i1 for j in range(VPR)]
        vm = [jnp.where(eq1[j], _NEG_INF, vs[j]) for j in range(VPR)]
        s2 = jnp.max(_max_tree(vm))
        i2 = jnp.min(_min_tree(
            [jnp.where(vm[j] == s2, iotas[j], _BIG_IDX) for j in range(VPR)]))
        for j in range(VPR):
            hit = jnp.logical_or(eq1[j], iotas[j] == i2)
            vout[pl.ds(off + LANES * j, LANES)] = jnp.where(hit, one, zero)
        return carry

    def chunk_body(ch, carry):
        base = (wid * ROWS_PER_WORKER + ch * CHUNK_ROWS) * PATH_NUM
        pltpu.sync_copy(score_hbm.at[pl.ds(base, CHUNK_ROWS * PATH_NUM)], vin)
        lax.fori_loop(0, CHUNK_ROWS, row_body, 0, unroll=4)
        pltpu.sync_copy(vout, out_hbm.at[pl.ds(base, CHUNK_ROWS * PATH_NUM)])
        return carry

    lax.fori_loop(0, NUM_CHUNKS, chunk_body, 0)


@jax.jit
def kernel(score):
    mesh = plsc.VectorSubcoreMesh(
        core_axis_name="c", subcore_axis_name="s",
        num_cores=NUM_CORES, num_subcores=NUM_SUBCORES)
    run = pl.kernel(
        _topk_body,
        out_type=jax.ShapeDtypeStruct((N_TOKENS * PATH_NUM,), jnp.int32),
        mesh=mesh,
        scratch_types=[
            pltpu.VMEM((CHUNK_ROWS * PATH_NUM,), jnp.float32),
            pltpu.VMEM((CHUNK_ROWS * PATH_NUM,), jnp.int32),
        ],
        compiler_params=pltpu.CompilerParams(needs_layout_passes=False),
    )
    flat = run(score.reshape(-1))
    return flat.reshape(N_TOKENS, PATH_NUM)


# speedup vs baseline: 3.1712x; 1.0409x over previous
import functools

import jax
import jax.numpy as jnp
from jax import lax
from jax.experimental import pallas as pl
from jax.experimental.pallas import tpu as pltpu
from jax.experimental.pallas import tpu_sc as plsc

PATH_NUM = 64
N_TOKENS = 32768
NUM_CORES = 2
NUM_SUBCORES = 16
NUM_WORKERS = NUM_CORES * NUM_SUBCORES
ROWS_PER_WORKER = N_TOKENS // NUM_WORKERS
CHUNK_ROWS = 256
NUM_CHUNKS = ROWS_PER_WORKER // CHUNK_ROWS
LANES = 16
VPR = PATH_NUM // LANES

_NEG_INF = float("-inf")
_BIG_IDX = PATH_NUM


def _min_tree(xs):
    return jnp.minimum(jnp.minimum(xs[0], xs[1]), jnp.minimum(xs[2], xs[3]))


def _max_tree(xs):
    return jnp.maximum(jnp.maximum(xs[0], xs[1]), jnp.maximum(xs[2], xs[3]))


def _topk_body(score_hbm, out_hbm, vin0, vin1, vout0, vout1,
               isem0, isem1, osem0, osem1):
    wid = lax.axis_index("s") * NUM_CORES + lax.axis_index("c")
    iotas = [lax.iota(jnp.int32, LANES) + LANES * j for j in range(VPR)]
    one = jnp.full((LANES,), 1, jnp.int32)
    zero = jnp.full((LANES,), 0, jnp.int32)
    vins = [vin0, vin1]
    vouts = [vout0, vout1]
    isems = [isem0, isem1]
    osems = [osem0, osem1]

    def make_row_body(vin, vout):
      def row_body(r, carry):
        off = r * PATH_NUM
        vs = [vin[pl.ds(off + LANES * j, LANES)] for j in range(VPR)]
        s1 = jnp.max(_max_tree(vs))
        i1 = jnp.min(_min_tree(
            [jnp.where(vs[j] == s1, iotas[j], _BIG_IDX) for j in range(VPR)]))
        eq1 = [iotas[j] == i1 for j in range(VPR)]
        vm = [jnp.where(eq1[j], _NEG_INF, vs[j]) for j in range(VPR)]
        s2 = jnp.max(_max_tree(vm))
        i2 = jnp.min(_min_tree(
            [jnp.where(vm[j] == s2, iotas[j], _BIG_IDX) for j in range(VPR)]))
        for j in range(VPR):
            hit = jnp.logical_or(eq1[j], iotas[j] == i2)
            vout[pl.ds(off + LANES * j, LANES)] = jnp.where(hit, one, zero)
        return carry
      return row_body

    def hbm_slice(ch):
        base = (wid * ROWS_PER_WORKER + ch * CHUNK_ROWS) * PATH_NUM
        return pl.ds(base, CHUNK_ROWS * PATH_NUM)

    out_handles = [None, None]
    pltpu.async_copy(score_hbm.at[hbm_slice(0)], vins[0], isems[0])
    for ch in range(NUM_CHUNKS):
        cur = ch % 2
        if ch + 1 < NUM_CHUNKS:
            nxt = (ch + 1) % 2
            pltpu.async_copy(score_hbm.at[hbm_slice(ch + 1)], vins[nxt],
                             isems[nxt])
        pltpu.make_async_copy(score_hbm.at[hbm_slice(ch)], vins[cur],
                              isems[cur]).wait()
        if out_handles[cur] is not None:
            out_handles[cur].wait()
        lax.fori_loop(0, CHUNK_ROWS, make_row_body(vins[cur], vouts[cur]), 0,
                      unroll=4)
        out_handles[cur] = pltpu.async_copy(
            vouts[cur], out_hbm.at[hbm_slice(ch)], osems[cur])
    for h in out_handles:
        if h is not None:
            h.wait()


@jax.jit
def kernel(score):
    mesh = plsc.VectorSubcoreMesh(
        core_axis_name="c", subcore_axis_name="s",
        num_cores=NUM_CORES, num_subcores=NUM_SUBCORES)
    run = pl.kernel(
        _topk_body,
        out_type=jax.ShapeDtypeStruct((N_TOKENS * PATH_NUM,), jnp.int32),
        mesh=mesh,
        scratch_types=[
            pltpu.VMEM((CHUNK_ROWS * PATH_NUM,), jnp.float32),
            pltpu.VMEM((CHUNK_ROWS * PATH_NUM,), jnp.float32),
            pltpu.VMEM((CHUNK_ROWS * PATH_NUM,), jnp.int32),
            pltpu.VMEM((CHUNK_ROWS * PATH_NUM,), jnp.int32),
            pltpu.SemaphoreType.DMA,
            pltpu.SemaphoreType.DMA,
            pltpu.SemaphoreType.DMA,
            pltpu.SemaphoreType.DMA,
        ],
        compiler_params=pltpu.CompilerParams(needs_layout_passes=False),
    )
    flat = run(score.reshape(-1))
    return flat.reshape(N_TOKENS, PATH_NUM)


# speedup vs baseline: 4.3830x; 1.3821x over previous
import functools

import jax
import jax.numpy as jnp
from jax import lax
from jax.experimental import pallas as pl
from jax.experimental.pallas import tpu as pltpu
from jax.experimental.pallas import tpu_sc as plsc

PATH_NUM = 64
N_TOKENS = 32768
NUM_CORES = 2
NUM_SUBCORES = 16
NUM_WORKERS = NUM_CORES * NUM_SUBCORES
ROWS_PER_WORKER = N_TOKENS // NUM_WORKERS
CHUNK_ROWS = 256
NUM_CHUNKS = ROWS_PER_WORKER // CHUNK_ROWS
LANES = 16
VPR = PATH_NUM // LANES
GROUP = 4

_NEG_INF = float("-inf")
_BIG_IDX = PATH_NUM


def _min_tree(xs):
    return jnp.minimum(jnp.minimum(xs[0], xs[1]), jnp.minimum(xs[2], xs[3]))


def _max_tree(xs):
    return jnp.maximum(jnp.maximum(xs[0], xs[1]), jnp.maximum(xs[2], xs[3]))


def _topk_body(score_hbm, out_hbm, vin0, vin1, vout0, vout1,
               isem0, isem1, osem0, osem1):
    wid = lax.axis_index("s") * NUM_CORES + lax.axis_index("c")
    iotas = [lax.iota(jnp.int32, LANES) + LANES * j for j in range(VPR)]
    one = jnp.full((LANES,), 1, jnp.int32)
    zero = jnp.full((LANES,), 0, jnp.int32)
    vins = [vin0, vin1]
    vouts = [vout0, vout1]
    isems = [isem0, isem1]
    osems = [osem0, osem1]

    def first_idx(eqs):
        cands = []
        for j in range(VPR):
            f = plsc.all_reduce_ffs(eqs[j])
            bad = jnp.logical_or(f > LANES - 1, f < 0)
            cands.append(jnp.where(bad, _BIG_IDX, f + LANES * j))
        return _min_tree(cands)

    def make_group_body(vin, vout):
      def group_body(i, carry):
        offs = [(i * GROUP + g) * PATH_NUM for g in range(GROUP)]
        vs = [[vin[pl.ds(offs[g] + LANES * j, LANES)] for j in range(VPR)]
              for g in range(GROUP)]
        s1 = [jnp.max(_max_tree(vs[g])) for g in range(GROUP)]
        eq1v = [[vs[g][j] == s1[g] for j in range(VPR)] for g in range(GROUP)]
        i1 = [first_idx(eq1v[g]) for g in range(GROUP)]
        eq1 = [[iotas[j] == i1[g] for j in range(VPR)] for g in range(GROUP)]
        vm = [[jnp.where(eq1[g][j], _NEG_INF, vs[g][j]) for j in range(VPR)]
              for g in range(GROUP)]
        s2 = [jnp.max(_max_tree(vm[g])) for g in range(GROUP)]
        eq2v = [[vm[g][j] == s2[g] for j in range(VPR)] for g in range(GROUP)]
        i2 = [first_idx(eq2v[g]) for g in range(GROUP)]
        for g in range(GROUP):
            for j in range(VPR):
                hit = jnp.logical_or(eq1[g][j], iotas[j] == i2[g])
                vout[pl.ds(offs[g] + LANES * j, LANES)] = \
                    jnp.where(hit, one, zero)
        return carry
      return group_body

    def hbm_slice(ch):
        base = (wid * ROWS_PER_WORKER + ch * CHUNK_ROWS) * PATH_NUM
        return pl.ds(base, CHUNK_ROWS * PATH_NUM)

    out_handles = [None, None]
    pltpu.async_copy(score_hbm.at[hbm_slice(0)], vins[0], isems[0])
    for ch in range(NUM_CHUNKS):
        cur = ch % 2
        if ch + 1 < NUM_CHUNKS:
            nxt = (ch + 1) % 2
            pltpu.async_copy(score_hbm.at[hbm_slice(ch + 1)], vins[nxt],
                             isems[nxt])
        pltpu.make_async_copy(score_hbm.at[hbm_slice(ch)], vins[cur],
                              isems[cur]).wait()
        if out_handles[cur] is not None:
            out_handles[cur].wait()
        lax.fori_loop(0, CHUNK_ROWS // GROUP,
                      make_group_body(vins[cur], vouts[cur]), 0)
        out_handles[cur] = pltpu.async_copy(
            vouts[cur], out_hbm.at[hbm_slice(ch)], osems[cur])
    for h in out_handles:
        if h is not None:
            h.wait()


@jax.jit
def kernel(score):
    mesh = plsc.VectorSubcoreMesh(
        core_axis_name="c", subcore_axis_name="s",
        num_cores=NUM_CORES, num_subcores=NUM_SUBCORES)
    run = pl.kernel(
        _topk_body,
        out_type=jax.ShapeDtypeStruct((N_TOKENS * PATH_NUM,), jnp.int32),
        mesh=mesh,
        scratch_types=[
            pltpu.VMEM((CHUNK_ROWS * PATH_NUM,), jnp.float32),
            pltpu.VMEM((CHUNK_ROWS * PATH_NUM,), jnp.float32),
            pltpu.VMEM((CHUNK_ROWS * PATH_NUM,), jnp.int32),
            pltpu.VMEM((CHUNK_ROWS * PATH_NUM,), jnp.int32),
            pltpu.SemaphoreType.DMA,
            pltpu.SemaphoreType.DMA,
            pltpu.SemaphoreType.DMA,
            pltpu.SemaphoreType.DMA,
        ],
        compiler_params=pltpu.CompilerParams(needs_layout_passes=False),
    )
    flat = run(score.reshape(-1))
    return flat.reshape(N_TOKENS, PATH_NUM)
